# baseline (speedup 1.0000x reference)
import jax
import jax.numpy as jnp
from jax import lax
from jax.experimental import pallas as pl

HEADS = 16
BUCKETS = 64
SEQ = 4096
DIM = 128
TOK = SEQ // BUCKETS
SL = 2


def _sortnet(mq, mk, qpos, kpos, wq, wk):
    sq = (jnp.dot(mq, wq[:DIM], preferred_element_type=jnp.float32)
          + jnp.dot(qpos, wq[DIM:], preferred_element_type=jnp.float32))
    sk = (jnp.dot(mk, wk[:DIM], preferred_element_type=jnp.float32)
          + jnp.dot(kpos, wk[DIM:], preferred_element_type=jnp.float32))
    r = lax.dot_general(sq, sk, (((1,), (1,)), ((), ())),
                        preferred_element_type=jnp.float32)
    r = r - jnp.max(r, axis=-1, keepdims=True)
    e = jnp.exp(r)
    return e / jnp.sum(e, axis=-1, keepdims=True)


def _body(q_ref, k_ref, qpos_ref, kpos_ref, wq_ref, wk_ref, out_ref):
    i = pl.program_id(0)
    for s in range(SL):
        h = lax.rem(i * SL + s, HEADS)
        mq = jnp.sum(q_ref[s].reshape(BUCKETS, TOK, DIM), axis=1) * (
            jnp.float32(1.0 / TOK))
        mk = jnp.sum(k_ref[s].reshape(BUCKETS, TOK, DIM), axis=1) * (
            jnp.float32(1.0 / TOK))
        out_ref[s] = _sortnet(mq, mk, qpos_ref[0, h], kpos_ref[0, h],
                              wq_ref[0, h], wk_ref[0, h])


def kernel(q, k, q_pos_emb, k_pos_emb, linear_sort_q, linear_sort_k):
    bh = q.shape[0]
    n = bh // SL
    return pl.pallas_call(
        _body,
        grid=(n,),
        in_specs=[
            pl.BlockSpec((SL, SEQ, DIM), lambda i: (i, 0, 0)),
            pl.BlockSpec((SL, SEQ, DIM), lambda i: (i, 0, 0)),
            pl.BlockSpec((1, HEADS, BUCKETS, DIM), lambda i: (0, 0, 0, 0)),
            pl.BlockSpec((1, HEADS, BUCKETS, DIM), lambda i: (0, 0, 0, 0)),
            pl.BlockSpec((1, HEADS, 2 * DIM, DIM), lambda i: (0, 0, 0, 0)),
            pl.BlockSpec((1, HEADS, 2 * DIM, DIM), lambda i: (0, 0, 0, 0)),
        ],
        out_specs=pl.BlockSpec((SL, BUCKETS, BUCKETS), lambda i: (i, 0, 0)),
        out_shape=jax.ShapeDtypeStruct((bh, BUCKETS, BUCKETS), jnp.float32),
    )(q, k, q_pos_emb, k_pos_emb, linear_sort_q, linear_sort_k)
